# baseline (speedup 1.0000x reference)
import jax
import jax.numpy as jnp
from jax import lax
from jax.experimental import pallas as pl
from jax.experimental.pallas import tpu as pltpu
from jax.experimental.pallas import tpu_sc as plsc

VOCAB = 100000
SEQ_LEN = 200
EMBED = 128
BATCH = 1024
EPS = 1e-5

NC = 2
NS = 16
NW = NC * NS
N_ROWS = BATCH * SEQ_LEN
NSLAB = 2
BATCH_S = BATCH // NSLAB
ROWS_S = BATCH_S * SEQ_LEN
ROWS_PER_TILE = ROWS_S // NW
CHUNK = 80
K = ROWS_PER_TILE // CHUNK

TC_B = 64


def _sc_gather_body(ids_hbm, table_hbm, out_hbm,
                    idx_v, buf_a, buf_b, gsem_a, gsem_b, osem_a, osem_b):
    wid = lax.axis_index("s") * NC + lax.axis_index("c")
    base_row = wid * ROWS_PER_TILE

    pltpu.sync_copy(ids_hbm.at[wid], idx_v)

    def fire_gather(k, buf, sem):
        pltpu.async_copy(table_hbm.at[idx_v.at[k]], buf, sem)

    def wait_gather(k, buf, sem):
        pltpu.make_async_copy(table_hbm.at[idx_v.at[k]], buf, sem).wait()

    def fire_scatter(k, buf, sem):
        pltpu.async_copy(buf, out_hbm.at[pl.ds(base_row + k * CHUNK, CHUNK)], sem)

    def wait_scatter(k, buf, sem):
        pltpu.make_async_copy(
            buf, out_hbm.at[pl.ds(base_row + k * CHUNK, CHUNK)], sem).wait()

    fire_gather(0, buf_a, gsem_a)
    fire_gather(1, buf_b, gsem_b)

    @pl.loop(0, K, step=2)
    def _chunk(k):
        wait_gather(k, buf_a, gsem_a)
        fire_scatter(k, buf_a, osem_a)

        wait_gather(k + 1, buf_b, gsem_b)
        fire_scatter(k + 1, buf_b, osem_b)

        wait_scatter(k, buf_a, osem_a)

        @pl.when(k + 2 < K)
        def _():
            fire_gather(k + 2, buf_a, gsem_a)

        wait_scatter(k + 1, buf_b, osem_b)

        @pl.when(k + 3 < K)
        def _():
            fire_gather(k + 3, buf_b, gsem_b)


def _tc_ln_body(x_ref, pos_ref, g_ref, b_ref, o_ref):
    t = x_ref[...] + pos_ref[...][None, :, :]
    mean = jnp.mean(t, axis=-1, keepdims=True)
    c = t - mean
    var = jnp.mean(c * c, axis=-1, keepdims=True)
    rstd = lax.rsqrt(var + EPS)
    o_ref[...] = c * rstd * g_ref[...] + b_ref[...]


@jax.jit
def _run(ids3d, token_table, pos_table, gamma, beta):
    mesh = plsc.VectorSubcoreMesh(core_axis_name="c", subcore_axis_name="s",
                                  num_cores=NC, num_subcores=NS)
    sc_gather = pl.kernel(
        _sc_gather_body,
        out_type=jax.ShapeDtypeStruct((ROWS_S, EMBED), jnp.float32),
        mesh=mesh,
        scratch_types=[
            pltpu.VMEM((K, CHUNK), jnp.int32),
            pltpu.VMEM((CHUNK, EMBED), jnp.float32),
            pltpu.VMEM((CHUNK, EMBED), jnp.float32),
            pltpu.SemaphoreType.DMA,
            pltpu.SemaphoreType.DMA,
            pltpu.SemaphoreType.DMA,
            pltpu.SemaphoreType.DMA,
        ],
    )

    tc_ln = pl.pallas_call(
        _tc_ln_body,
        out_shape=jax.ShapeDtypeStruct((BATCH_S, SEQ_LEN, EMBED), jnp.float32),
        grid=(BATCH_S // TC_B,),
        in_specs=[
            pl.BlockSpec((TC_B, SEQ_LEN, EMBED), lambda i: (i, 0, 0)),
            pl.BlockSpec((SEQ_LEN, EMBED), lambda i: (0, 0)),
            pl.BlockSpec((EMBED,), lambda i: (0,)),
            pl.BlockSpec((EMBED,), lambda i: (0,)),
        ],
        out_specs=pl.BlockSpec((TC_B, SEQ_LEN, EMBED), lambda i: (i, 0, 0)),
    )

    outs = []
    for s in range(NSLAB):
        g = sc_gather(ids3d[s], token_table)
        x = g.reshape(BATCH_S, SEQ_LEN, EMBED)
        outs.append(tc_ln(x, pos_table, gamma, beta))
    return jnp.concatenate(outs, axis=0)


def kernel(input_ids, token_table, pos_table, gamma, beta):
    ids3d = jnp.reshape(input_ids.astype(jnp.int32), (NSLAB, NW, K, CHUNK))
    return _run(ids3d, token_table, pos_table, gamma, beta)


# speedup vs baseline: 1.4053x; 1.4053x over previous
import jax
import jax.numpy as jnp
from jax import lax
from jax.experimental import pallas as pl
from jax.experimental.pallas import tpu as pltpu
from jax.experimental.pallas import tpu_sc as plsc

VOCAB = 100000
SEQ_LEN = 200
EMBED = 128
BATCH = 1024
EPS = 1e-5

NC = 2
NS = 16
NW = NC * NS
N_ROWS = BATCH * SEQ_LEN
ROWS_PER_TILE = N_ROWS // NW
CHUNK = 64
K = ROWS_PER_TILE // CHUNK
NBUF = 4

TC_B = 64


def _sc_gather_body(ids_hbm, table_hbm, out_hbm, idx_v, *scratch):
    bufs = scratch[:NBUF]
    gsems = scratch[NBUF:2 * NBUF]
    osems = scratch[2 * NBUF:3 * NBUF]

    wid = lax.axis_index("s") * NC + lax.axis_index("c")
    base_row = wid * ROWS_PER_TILE

    pltpu.sync_copy(ids_hbm.at[wid], idx_v)

    def fire_gather(k, j):
        pltpu.async_copy(table_hbm.at[idx_v.at[k]], bufs[j], gsems[j])

    def wait_gather(k, j):
        pltpu.make_async_copy(table_hbm.at[idx_v.at[k]], bufs[j], gsems[j]).wait()

    def fire_scatter(k, j):
        pltpu.async_copy(
            bufs[j], out_hbm.at[pl.ds(base_row + k * CHUNK, CHUNK)], osems[j])

    def wait_scatter(k, j):
        pltpu.make_async_copy(
            bufs[j], out_hbm.at[pl.ds(base_row + k * CHUNK, CHUNK)],
            osems[j]).wait()

    for j in range(NBUF):
        fire_gather(j, j)

    @pl.loop(0, K, step=NBUF)
    def _chunk(k):
        for j in range(NBUF):
            wait_gather(k + j, j)
            fire_scatter(k + j, j)
        for j in range(NBUF):
            wait_scatter(k + j, j)

            @pl.when(k + NBUF + j < K)
            def _():
                fire_gather(k + NBUF + j, j)


def _tc_ln_body(x_ref, pos_ref, g_ref, b_ref, o_ref):
    t = x_ref[...] + pos_ref[...][None, :, :]
    mean = jnp.mean(t, axis=-1, keepdims=True)
    c = t - mean
    var = jnp.mean(c * c, axis=-1, keepdims=True)
    rstd = lax.rsqrt(var + EPS)
    o_ref[...] = c * rstd * g_ref[...] + b_ref[...]


@jax.jit
def _run(ids3d, token_table, pos_table, gamma, beta):
    mesh = plsc.VectorSubcoreMesh(core_axis_name="c", subcore_axis_name="s",
                                  num_cores=NC, num_subcores=NS)
    gathered = pl.kernel(
        _sc_gather_body,
        out_type=jax.ShapeDtypeStruct((N_ROWS, EMBED), jnp.float32),
        mesh=mesh,
        scratch_types=(
            [pltpu.VMEM((K, CHUNK), jnp.int32)]
            + [pltpu.VMEM((CHUNK, EMBED), jnp.float32) for _ in range(NBUF)]
            + [pltpu.SemaphoreType.DMA for _ in range(2 * NBUF)]
        ),
    )(ids3d, token_table)

    x = gathered.reshape(BATCH, SEQ_LEN, EMBED)
    out = pl.pallas_call(
        _tc_ln_body,
        out_shape=jax.ShapeDtypeStruct((BATCH, SEQ_LEN, EMBED), jnp.float32),
        grid=(BATCH // TC_B,),
        in_specs=[
            pl.BlockSpec((TC_B, SEQ_LEN, EMBED), lambda i: (i, 0, 0)),
            pl.BlockSpec((SEQ_LEN, EMBED), lambda i: (0, 0)),
            pl.BlockSpec((EMBED,), lambda i: (0,)),
            pl.BlockSpec((EMBED,), lambda i: (0,)),
        ],
        out_specs=pl.BlockSpec((TC_B, SEQ_LEN, EMBED), lambda i: (i, 0, 0)),
    )(x, pos_table, gamma, beta)
    return out


def kernel(input_ids, token_table, pos_table, gamma, beta):
    ids3d = jnp.reshape(input_ids.astype(jnp.int32), (NW, K, CHUNK))
    return _run(ids3d, token_table, pos_table, gamma, beta)


# speedup vs baseline: 1.4072x; 1.0013x over previous
import jax
import jax.numpy as jnp
from jax import lax
from jax.experimental import pallas as pl
from jax.experimental.pallas import tpu as pltpu
from jax.experimental.pallas import tpu_sc as plsc

VOCAB = 100000
SEQ_LEN = 200
EMBED = 128
BATCH = 1024
EPS = 1e-5

NC = 2
NS = 16
NW = NC * NS
N_ROWS = BATCH * SEQ_LEN
ROWS_PER_TILE = N_ROWS // NW
CHUNK = 32
K = ROWS_PER_TILE // CHUNK
NBUF = 8

TC_B = 64


def _sc_gather_body(ids_hbm, table_hbm, out_hbm, idx_v, *scratch):
    bufs = scratch[:NBUF]
    gsems = scratch[NBUF:2 * NBUF]
    osems = scratch[2 * NBUF:3 * NBUF]

    wid = lax.axis_index("s") * NC + lax.axis_index("c")
    base_row = wid * ROWS_PER_TILE

    pltpu.sync_copy(ids_hbm.at[wid], idx_v)

    def fire_gather(k, j):
        pltpu.async_copy(table_hbm.at[idx_v.at[k]], bufs[j], gsems[j])

    def wait_gather(k, j):
        pltpu.make_async_copy(table_hbm.at[idx_v.at[k]], bufs[j], gsems[j]).wait()

    def fire_scatter(k, j):
        pltpu.async_copy(
            bufs[j], out_hbm.at[pl.ds(base_row + k * CHUNK, CHUNK)], osems[j])

    def wait_scatter(k, j):
        pltpu.make_async_copy(
            bufs[j], out_hbm.at[pl.ds(base_row + k * CHUNK, CHUNK)],
            osems[j]).wait()

    for j in range(NBUF):
        fire_gather(j, j)

    @pl.loop(0, K, step=NBUF)
    def _chunk(k):
        for j in range(NBUF):
            wait_gather(k + j, j)
            fire_scatter(k + j, j)
        for j in range(NBUF):
            wait_scatter(k + j, j)

            @pl.when(k + NBUF + j < K)
            def _():
                fire_gather(k + NBUF + j, j)


def _tc_ln_body(x_ref, pos_ref, g_ref, b_ref, o_ref):
    t = x_ref[...] + pos_ref[...][None, :, :]
    mean = jnp.mean(t, axis=-1, keepdims=True)
    c = t - mean
    var = jnp.mean(c * c, axis=-1, keepdims=True)
    rstd = lax.rsqrt(var + EPS)
    o_ref[...] = c * rstd * g_ref[...] + b_ref[...]


@jax.jit
def _run(ids3d, token_table, pos_table, gamma, beta):
    mesh = plsc.VectorSubcoreMesh(core_axis_name="c", subcore_axis_name="s",
                                  num_cores=NC, num_subcores=NS)
    gathered = pl.kernel(
        _sc_gather_body,
        out_type=jax.ShapeDtypeStruct((N_ROWS, EMBED), jnp.float32),
        mesh=mesh,
        scratch_types=(
            [pltpu.VMEM((K, CHUNK), jnp.int32)]
            + [pltpu.VMEM((CHUNK, EMBED), jnp.float32) for _ in range(NBUF)]
            + [pltpu.SemaphoreType.DMA for _ in range(2 * NBUF)]
        ),
    )(ids3d, token_table)

    x = gathered.reshape(BATCH, SEQ_LEN, EMBED)
    out = pl.pallas_call(
        _tc_ln_body,
        out_shape=jax.ShapeDtypeStruct((BATCH, SEQ_LEN, EMBED), jnp.float32),
        grid=(BATCH // TC_B,),
        in_specs=[
            pl.BlockSpec((TC_B, SEQ_LEN, EMBED), lambda i: (i, 0, 0)),
            pl.BlockSpec((SEQ_LEN, EMBED), lambda i: (0, 0)),
            pl.BlockSpec((EMBED,), lambda i: (0,)),
            pl.BlockSpec((EMBED,), lambda i: (0,)),
        ],
        out_specs=pl.BlockSpec((TC_B, SEQ_LEN, EMBED), lambda i: (i, 0, 0)),
    )(x, pos_table, gamma, beta)
    return out


def kernel(input_ids, token_table, pos_table, gamma, beta):
    ids3d = jnp.reshape(input_ids.astype(jnp.int32), (NW, K, CHUNK))
    return _run(ids3d, token_table, pos_table, gamma, beta)
